# Initial kernel scaffold; baseline (speedup 1.0000x reference)
import jax
import jax.numpy as jnp
from jax.experimental import pallas as pl


def kernel(memorized_embedding, query_src_emb, query_rel_emb, Wq, Wk, W_bs, b_bs, edge_src, edge_dst, eg_ids, segment_ids):
    raise NotImplementedError("write your pallas kernel here")


# speedup vs baseline: 1.5262x; 1.5262x over previous
import functools

import jax
import jax.numpy as jnp
from jax import lax
from jax.experimental import pallas as pl
from jax.experimental.pallas import tpu as pltpu, tpu_sc as plsc

N = 10000
E = 320000
D = 128
B = 16
DE = D + B
NSEG = 10240
NW = 32
CHUNK = E // NW
ROWBLK = 2000
HI = jax.lax.Precision.HIGHEST


def _bf(x):
    return x.astype(jnp.bfloat16).astype(jnp.float32)


def _split_mm(x, y, dims):
    xh = _bf(x); xl = x - xh
    yh = _bf(y); yl = y - yh
    dg = lambda a, b: lax.dot_general(a, b, (dims, ((), ())), precision=HI,
                                      preferred_element_type=jnp.float32)
    return ((dg(xh, yh) + dg(xl, yl)) + (dg(xh, yl) + dg(xl, yh)))


def _half_mm(xb, y, dims):
    yh = _bf(y); yl = y - yh
    dg = lambda a, b: lax.dot_general(a, b, (dims, ((), ())), precision=HI,
                                      preferred_element_type=jnp.float32)
    return dg(xb, yh) + dg(xb, yl)


def _kw_body(wq03, wk03, wq1, wq2, wk1, wk2, qs, qr, a_o, av_o, bv_o, c_o):
    a_o[...] = _split_mm(wq03[...], wk03[...], ((0,), (0,)))
    t = (lax.dot_general(qs[...], wk1[...], (((1,), (1,)), ((), ())), precision=HI)
         + lax.dot_general(qr[...], wk2[...], (((1,), (1,)), ((), ())), precision=HI))
    u = (lax.dot_general(qs[...], wq1[...], (((1,), (1,)), ((), ())), precision=HI)
         + lax.dot_general(qr[...], wq2[...], (((1,), (1,)), ((), ())), precision=HI))
    av_o[...] = _split_mm(t, wq03[...], ((1,), (0,)))
    bv_o[...] = _split_mm(u, wk03[...], ((1,), (0,)))
    c_o[...] = jnp.sum(u * t, axis=-1).reshape(1, B)


def _make_kw(interpret=False):
    return pl.pallas_call(
        _kw_body,
        out_shape=[
            jax.ShapeDtypeStruct((D, D), jnp.float32),
            jax.ShapeDtypeStruct((B, D), jnp.float32),
            jax.ShapeDtypeStruct((B, D), jnp.float32),
            jax.ShapeDtypeStruct((1, B), jnp.float32),
        ],
        interpret=interpret,
    )


def _k0_body(mem, a, av, bv, crow, zext_o, mext_o):
    m = mem[...]
    z = _half_mm(m, a[...], ((1,), (0,)))
    u = _half_mm(m, av[...], ((1,), (1,)))
    v = _half_mm(m, bv[...], ((1,), (1,))) + crow[...]
    zext_o[...] = jnp.concatenate([z, u], axis=-1)
    mext_o[...] = jnp.concatenate([m, v], axis=-1)


def _make_k0(interpret=False):
    return pl.pallas_call(
    _k0_body,
    interpret=interpret,
    grid=(N // ROWBLK,),
    in_specs=[
        pl.BlockSpec((ROWBLK, D), lambda i: (i, 0)),
        pl.BlockSpec((D, D), lambda i: (0, 0)),
        pl.BlockSpec((B, D), lambda i: (0, 0)),
        pl.BlockSpec((B, D), lambda i: (0, 0)),
        pl.BlockSpec((1, B), lambda i: (0, 0)),
    ],
    out_specs=[
        pl.BlockSpec((ROWBLK, DE), lambda i: (i, 0)),
        pl.BlockSpec((ROWBLK, DE), lambda i: (i, 0)),
    ],
    out_shape=[
        jax.ShapeDtypeStruct((N, DE), jnp.float32),
        jax.ShapeDtypeStruct((N, DE), jnp.float32),
    ],
    )


def _kf_body(wcol, mem, wbs, brow, out_o):
    agg = (wcol[...] * mem[...]).astype(jnp.bfloat16)
    x = lax.dot_general(agg, wbs[...].astype(jnp.bfloat16), (((1,), (1,)), ((), ())),
                        preferred_element_type=jnp.float32)
    x = x + brow[...]
    out_o[...] = jnp.where(x > 0, x, 0.01 * x)


def _make_kf(interpret=False):
    return pl.pallas_call(
        _kf_body,
        interpret=interpret,
        grid=(N // ROWBLK,),
        in_specs=[
            pl.BlockSpec((ROWBLK, 1), lambda i: (i, 0)),
            pl.BlockSpec((ROWBLK, D), lambda i: (i, 0)),
            pl.BlockSpec((D, D), lambda i: (0, 0)),
            pl.BlockSpec((1, D), lambda i: (0, 0)),
        ],
        out_specs=pl.BlockSpec((ROWBLK, D), lambda i: (i, 0)),
        out_shape=jax.ShapeDtypeStruct((N, D), jnp.float32),
    )


_kw = _make_kw()
_k0 = _make_k0()
_kf = _make_kf()


def kernel(memorized_embedding, query_src_emb, query_rel_emb, Wq, Wk, W_bs, b_bs,
           edge_src, edge_dst, eg_ids, segment_ids):
    bf = lambda x: lax.optimization_barrier(x.astype(jnp.bfloat16)).astype(jnp.float32)
    mem = memorized_embedding
    memb = bf(mem)
    wqb = [bf(Wq[:, i * D:(i + 1) * D]) for i in range(4)]
    wkb = [bf(Wk[:, i * D:(i + 1) * D]) for i in range(4)]
    a, av, bv, crow = _kw(wqb[0] + wqb[3], wkb[0] + wkb[3], wqb[1], wqb[2],
                          wkb[1], wkb[2], bf(query_src_emb), bf(query_rel_emb))
    zext, mext = _k0(memb, a, av, bv, crow)

    zr = lax.optimization_barrier(zext[edge_src])
    mr = lax.optimization_barrier(mext[edge_dst])
    prod = lax.optimization_barrier(zr[:, :D] * mr[:, :D])
    logits = (jnp.sum(prod, axis=-1)
              + zr[jnp.arange(E), D + eg_ids] + mr[jnp.arange(E), D + eg_ids])
    seg_max = jax.ops.segment_max(logits, segment_ids, num_segments=N)
    seg_max = jnp.where(jnp.isfinite(seg_max), seg_max, 0.0)
    ex = jnp.exp(logits - seg_max[segment_ids])
    den = jax.ops.segment_sum(ex, segment_ids, num_segments=N)
    att = ex / den[segment_ids]
    w = jax.ops.segment_sum(att, edge_dst, num_segments=N)

    updated = _kf(w.reshape(N, 1), mem, W_bs, b_bs.reshape(1, D))
    return (updated, att)
